# Initial kernel scaffold; baseline (speedup 1.0000x reference)
import jax
import jax.numpy as jnp
from jax.experimental import pallas as pl


def kernel(hidden_states, input_norm_w, post_norm_w, W_dq, norm_q_w, W_uq, W_dkv, norm_kv_w, W_ukv, W_o, W_gate, Wg_shared, Wu_shared, Wd_shared, Wg_experts, Wu_experts, Wd_experts):
    raise NotImplementedError("write your pallas kernel here")


# speedup vs baseline: 1.5739x; 1.5739x over previous
import functools

import numpy as np
import jax
import jax.numpy as jnp
from jax.experimental import pallas as pl

H = 1024; I = 512; NH = 16; DQ = 384; DKV = 128; DH = 64; DR = 32
E = 16; K = 2; MAXLEN = 4096; S = 2048; B = 1
EPS = 1.1920929e-07
MB = 256


def _rope_tables():
    inv_freq = 1.0 / (10000.0 ** (np.arange(0, DR, 2, dtype=np.float32) / DR))
    t = np.arange(S, dtype=np.float32)
    freqs = np.outer(t, inv_freq)
    emb = np.concatenate([freqs, freqs], axis=-1)
    return jnp.asarray(np.cos(emb)), jnp.asarray(np.sin(emb))


def _rms(x, w):
    return x * jax.lax.rsqrt(jnp.mean(x * x, axis=-1, keepdims=True) + EPS) * w


def _silu(x):
    return x * jax.nn.sigmoid(x)


def _dot(a, b):
    return jnp.dot(a, b, preferred_element_type=jnp.float32)


def _pre_attn_kernel(hs_ref, inw_ref, wdq_ref, nqw_ref, wuq_ref, wdkv_ref,
                     nkvw_ref, wukv_ref, q_ref, kv_ref, kr_ref):
    x = _rms(hs_ref[...], inw_ref[...])
    cq = _dot(x, wdq_ref[...])
    q_ref[...] = _dot(_rms(cq, nqw_ref[...]), wuq_ref[...])
    ckv = _dot(x, wdkv_ref[...])
    kv_ref[...] = _dot(_rms(ckv[:, :DKV], nkvw_ref[...]), wukv_ref[...])
    kr_ref[...] = ckv[:, DKV:]


def _rope_apply(x, cos, sin):
    x1 = x[:, : DR // 2]
    x2 = x[:, DR // 2:]
    rot = jnp.concatenate([-x2, x1], axis=-1)
    return x * cos + rot * sin


def _attn_kernel(q_ref, kv_ref, kr_ref, cos_ref, sin_ref, o_ref):
    qh = q_ref[0]
    kvh = kv_ref[0]
    cos = cos_ref[...]
    sin = sin_ref[...]
    q_r = _rope_apply(qh[:, DH:], cos, sin)
    k_r = _rope_apply(kr_ref[...], cos, sin)
    q = jnp.concatenate([qh[:, :DH], q_r], axis=-1)
    k = jnp.concatenate([kvh[:, :DH], k_r], axis=-1)
    scale = 1.0 / np.sqrt(np.float32(DH + DR))
    s = _dot(q, k.T) * scale
    m = jnp.max(s, axis=-1, keepdims=True)
    p = jnp.exp(s - m)
    p = p / jnp.sum(p, axis=-1, keepdims=True)
    o_ref[0] = _dot(p, kvh[:, DH:])


def _post_kernel(o_ref, hs_ref, wo_ref, pnw_ref, wgs_ref, wus_ref, wds_ref,
                 wg_ref, ybase_ref, x2_ref, gate_ref):
    attn_out = _dot(o_ref[...], wo_ref[...]) + hs_ref[...]
    x2 = _rms(attn_out, pnw_ref[...])
    x2_ref[...] = x2
    shared = _dot(_silu(_dot(x2, wgs_ref[...])) * _dot(x2, wus_ref[...]),
                  wds_ref[...])
    ybase_ref[...] = attn_out + shared
    scores = jax.nn.sigmoid(_dot(x2, wg_ref[...]))
    lane = jax.lax.broadcasted_iota(jnp.int32, scores.shape, 1)
    m1 = jnp.max(scores, axis=-1, keepdims=True)
    i1 = jnp.min(jnp.where(scores >= m1, lane, E), axis=-1, keepdims=True)
    first1 = lane == i1
    masked = jnp.where(first1, -jnp.inf, scores)
    m2 = jnp.max(masked, axis=-1, keepdims=True)
    i2 = jnp.min(jnp.where(masked >= m2, lane, E), axis=-1, keepdims=True)
    first2 = lane == i2
    denom = m1 + m2
    gate_ref[...] = jnp.where(first1, m1 / denom, 0.0) + \
        jnp.where(first2, m2 / denom, 0.0)


def _experts_kernel(x2_ref, gate_ref, ybase_ref, wge_ref, wue_ref, wde_ref,
                    out_ref):
    e = pl.program_id(0)
    x = x2_ref[...]
    lane = jax.lax.broadcasted_iota(jnp.int32, (1, E), 1)
    g = jnp.sum(gate_ref[...] * (lane == e).astype(jnp.float32), axis=-1,
                keepdims=True)
    h = _silu(_dot(x, wge_ref[0])) * _dot(x, wue_ref[0])
    contrib = _dot(h, wde_ref[0]) * g

    @pl.when(e == 0)
    def _():
        out_ref[...] = ybase_ref[...] + contrib

    @pl.when(e != 0)
    def _():
        out_ref[...] += contrib


def kernel(hidden_states, input_norm_w, post_norm_w, W_dq, norm_q_w, W_uq,
           W_dkv, norm_kv_w, W_ukv, W_o, W_gate, Wg_shared, Wu_shared,
           Wd_shared, Wg_experts, Wu_experts, Wd_experts):
    hs = hidden_states.reshape(S, H)
    cos, sin = _rope_tables()
    f32 = jnp.float32

    inw = input_norm_w.reshape(1, H)
    nqw = norm_q_w.reshape(1, DQ)
    nkvw = norm_kv_w.reshape(1, DKV)
    pnw = post_norm_w.reshape(1, H)

    nm = S // MB
    q_all, kv_all, kr_all = pl.pallas_call(
        _pre_attn_kernel,
        grid=(nm,),
        in_specs=[
            pl.BlockSpec((MB, H), lambda m: (m, 0)),
            pl.BlockSpec((1, H), lambda m: (0, 0)),
            pl.BlockSpec((H, DQ), lambda m: (0, 0)),
            pl.BlockSpec((1, DQ), lambda m: (0, 0)),
            pl.BlockSpec((DQ, NH * (DH + DR)), lambda m: (0, 0)),
            pl.BlockSpec((H, DKV + DR), lambda m: (0, 0)),
            pl.BlockSpec((1, DKV), lambda m: (0, 0)),
            pl.BlockSpec((DKV, NH * 2 * DH), lambda m: (0, 0)),
        ],
        out_specs=[
            pl.BlockSpec((MB, NH * (DH + DR)), lambda m: (m, 0)),
            pl.BlockSpec((MB, NH * 2 * DH), lambda m: (m, 0)),
            pl.BlockSpec((MB, DR), lambda m: (m, 0)),
        ],
        out_shape=[
            jax.ShapeDtypeStruct((S, NH * (DH + DR)), f32),
            jax.ShapeDtypeStruct((S, NH * 2 * DH), f32),
            jax.ShapeDtypeStruct((S, DR), f32),
        ],
    )(hs, inw, W_dq, nqw, W_uq, W_dkv, nkvw, W_ukv)

    q_heads = q_all.reshape(S, NH, DH + DR).transpose(1, 0, 2)
    kv_heads = kv_all.reshape(S, NH, 2 * DH).transpose(1, 0, 2)

    o_heads = pl.pallas_call(
        _attn_kernel,
        grid=(NH,),
        in_specs=[
            pl.BlockSpec((1, S, DH + DR), lambda h: (h, 0, 0)),
            pl.BlockSpec((1, S, 2 * DH), lambda h: (h, 0, 0)),
            pl.BlockSpec((S, DR), lambda h: (0, 0)),
            pl.BlockSpec((S, DR), lambda h: (0, 0)),
            pl.BlockSpec((S, DR), lambda h: (0, 0)),
        ],
        out_specs=pl.BlockSpec((1, S, DH), lambda h: (h, 0, 0)),
        out_shape=jax.ShapeDtypeStruct((NH, S, DH), f32),
    )(q_heads, kv_heads, kr_all, cos, sin)

    o_flat = o_heads.transpose(1, 0, 2).reshape(S, NH * DH)

    y_base, x2, gate_dense = pl.pallas_call(
        _post_kernel,
        grid=(nm,),
        in_specs=[
            pl.BlockSpec((MB, NH * DH), lambda m: (m, 0)),
            pl.BlockSpec((MB, H), lambda m: (m, 0)),
            pl.BlockSpec((NH * DH, H), lambda m: (0, 0)),
            pl.BlockSpec((1, H), lambda m: (0, 0)),
            pl.BlockSpec((H, I), lambda m: (0, 0)),
            pl.BlockSpec((H, I), lambda m: (0, 0)),
            pl.BlockSpec((I, H), lambda m: (0, 0)),
            pl.BlockSpec((H, E), lambda m: (0, 0)),
        ],
        out_specs=[
            pl.BlockSpec((MB, H), lambda m: (m, 0)),
            pl.BlockSpec((MB, H), lambda m: (m, 0)),
            pl.BlockSpec((MB, E), lambda m: (m, 0)),
        ],
        out_shape=[
            jax.ShapeDtypeStruct((S, H), f32),
            jax.ShapeDtypeStruct((S, H), f32),
            jax.ShapeDtypeStruct((S, E), f32),
        ],
    )(o_flat, hs, W_o, pnw, Wg_shared, Wu_shared, Wd_shared, W_gate)

    out = pl.pallas_call(
        _experts_kernel,
        grid=(E,),
        in_specs=[
            pl.BlockSpec((S, H), lambda e: (0, 0)),
            pl.BlockSpec((S, E), lambda e: (0, 0)),
            pl.BlockSpec((S, H), lambda e: (0, 0)),
            pl.BlockSpec((1, H, I), lambda e: (e, 0, 0)),
            pl.BlockSpec((1, H, I), lambda e: (e, 0, 0)),
            pl.BlockSpec((1, I, H), lambda e: (e, 0, 0)),
        ],
        out_specs=pl.BlockSpec((S, H), lambda e: (0, 0)),
        out_shape=jax.ShapeDtypeStruct((S, H), f32),
    )(x2, gate_dense, y_base, Wg_experts, Wu_experts, Wd_experts)

    return out.reshape(B, S, H)
